# baseline (speedup 1.0000x reference)
import functools

import jax
import jax.numpy as jnp
from jax.experimental import pallas as pl

_PREC = jax.lax.Precision.DEFAULT


def _mm_body(a_ref, w_ref, b_ref, o_ref, *, relu):
    acc = jax.lax.dot_general(
        a_ref[...], w_ref[...], (((1,), (0,)), ((), ())),
        precision=_PREC, preferred_element_type=jnp.float32)
    acc = acc + b_ref[...]
    if relu:
        acc = jnp.maximum(acc, 0.0)
    o_ref[...] = acc


def _mm(a, w, b, relu=False, block_m=512):
    M, K = a.shape
    N = w.shape[1]
    assert M % block_m == 0, (M, block_m)
    return pl.pallas_call(
        functools.partial(_mm_body, relu=relu),
        grid=(M // block_m,),
        in_specs=[
            pl.BlockSpec((block_m, K), lambda i: (i, 0)),
            pl.BlockSpec((K, N), lambda i: (0, 0)),
            pl.BlockSpec((1, N), lambda i: (0, 0)),
        ],
        out_specs=pl.BlockSpec((block_m, N), lambda i: (i, 0)),
        out_shape=jax.ShapeDtypeStruct((M, N), jnp.float32),
    )(a, w, b.reshape(1, N))


def _vq_body(z_ref, zz_ref, ct_ref, c2_ref, cb_ref, o_ref):
    z = z_ref[...]
    s = jax.lax.dot_general(
        z, ct_ref[...], (((1,), (0,)), ((), ())),
        precision=_PREC, preferred_element_type=jnp.float32)
    d = (zz_ref[...] - 2.0 * s) + c2_ref[...]
    codes = jnp.argmin(d, axis=1)
    bm, K = d.shape
    onehot = (jax.lax.broadcasted_iota(jnp.int32, (bm, K), 1)
              == codes[:, None]).astype(jnp.float32)
    o_ref[...] = jax.lax.dot_general(
        onehot, cb_ref[...], (((1,), (0,)), ((), ())),
        precision=jax.lax.Precision.HIGHEST, preferred_element_type=jnp.float32)


def _vq(z, codebook, block_m=512):
    M, D = z.shape
    K = codebook.shape[0]
    ct = codebook.T
    c2 = jnp.sum(codebook * codebook, axis=1).reshape(1, K)
    zz = jnp.sum(z * z, axis=1, keepdims=True)
    return pl.pallas_call(
        _vq_body,
        grid=(M // block_m,),
        in_specs=[
            pl.BlockSpec((block_m, D), lambda i: (i, 0)),
            pl.BlockSpec((block_m, 1), lambda i: (i, 0)),
            pl.BlockSpec((D, K), lambda i: (0, 0)),
            pl.BlockSpec((1, K), lambda i: (0, 0)),
            pl.BlockSpec((K, D), lambda i: (0, 0)),
        ],
        out_specs=pl.BlockSpec((block_m, D), lambda i: (i, 0)),
        out_shape=jax.ShapeDtypeStruct((M, D), jnp.float32),
    )(z, zz, ct, c2, codebook)


def _im2col_s2(img, kh=4, kw=4):
    B, Hp, Wp, C = img.shape
    Ho = (Hp - kh) // 2 + 1
    Wo = (Wp - kw) // 2 + 1
    taps = [img[:, ti:ti + 2 * Ho - 1:2, tj:tj + 2 * Wo - 1:2, :]
            for ti in range(kh) for tj in range(kw)]
    p = jnp.stack(taps, axis=3)
    return p.reshape(B * Ho * Wo, kh * kw * C)


def kernel(x, enc_w1, enc_b1, enc_w2, enc_b2, codebook,
           dec_w1, dec_b1, dec_w2, dec_b2):
    B, Cin, H, W = x.shape
    hidden = enc_w1.shape[0]
    D = enc_w2.shape[0]
    H1, W1 = H // 2, W // 2
    H2, W2 = H1 // 2, W1 // 2

    xp = jnp.pad(jnp.transpose(x, (0, 2, 3, 1)),
                 ((0, 0), (1, 1), (1, 1), (0, 0)))
    p1 = _im2col_s2(xp)
    w1 = enc_w1.transpose(2, 3, 1, 0).reshape(16 * Cin, hidden)
    h = _mm(p1, w1, enc_b1, relu=True)

    hp = jnp.pad(h.reshape(B, H1, W1, hidden),
                 ((0, 0), (1, 1), (1, 1), (0, 0)))
    p2 = _im2col_s2(hp)
    w2 = enc_w2.transpose(2, 3, 1, 0).reshape(16 * hidden, D)
    z_e = _mm(p2, w2, enc_b2)

    z_q = _vq(z_e, codebook)

    zp = jnp.pad(z_q.reshape(B, H2, W2, D),
                 ((0, 0), (1, 1), (1, 1), (0, 0)))
    wd1 = dec_w1.transpose(2, 3, 1, 0)
    g_ph = []
    for a in (0, 1):
        for b in (0, 1):
            taps = [zp[:, a + u:a + u + H2, b + v:b + v + W2, :]
                    for u in (0, 1) for v in (0, 1)]
            pab = jnp.stack(taps, axis=3).reshape(B * H2 * W2, 4 * D)
            wab = wd1[a::2, b::2].reshape(4 * D, hidden)
            g_ph.append(_mm(pab, wab, dec_b1, relu=True))
    g = jnp.stack(g_ph, 0).reshape(2, 2, B, H2, W2, hidden)
    g = g.transpose(2, 3, 0, 4, 1, 5).reshape(B, H1, W1, hidden)

    gp = jnp.pad(g, ((0, 0), (1, 1), (1, 1), (0, 0)))
    wd2 = dec_w2.transpose(2, 3, 1, 0)
    NPAD = 128
    b2p = jnp.pad(dec_b2, (0, NPAD - Cin))
    o_ph = []
    for a in (0, 1):
        for b in (0, 1):
            taps = [gp[:, a + u:a + u + H1, b + v:b + v + W1, :]
                    for u in (0, 1) for v in (0, 1)]
            pab = jnp.stack(taps, axis=3).reshape(B * H1 * W1, 4 * hidden)
            wab = wd2[a::2, b::2].reshape(4 * hidden, Cin)
            wab = jnp.pad(wab, ((0, 0), (0, NPAD - Cin)))
            o_ph.append(_mm(pab, wab, b2p)[:, :Cin])
    out = jnp.stack(o_ph, 0).reshape(2, 2, B, H1, W1, Cin)
    out = out.transpose(2, 3, 0, 4, 1, 5).reshape(B, H, W, Cin)
    return out.transpose(0, 3, 1, 2)


# speedup vs baseline: 12.2774x; 12.2774x over previous
import functools

import jax
import jax.numpy as jnp
from jax.experimental import pallas as pl
from jax.experimental.pallas import tpu as pltpu

_DEF = jax.lax.Precision.DEFAULT


def _dot(a, w, prec=_DEF):
    return jax.lax.dot_general(a, w, (((1,), (0,)), ((), ())),
                               precision=prec,
                               preferred_element_type=jnp.float32)


def _body(xs_ref, w1_ref, b1_ref, w2_ref, b2_ref, ct_ref, c2_ref, cb_ref,
          wd1_ref, bd1_ref, wd2_ref, bd2_ref, o_ref,
          hcp_ref, zqp_ref, gp_ref):
    H1 = 112
    H2 = 56
    HID = w1_ref.shape[2]
    D = w2_ref.shape[2]
    K = cb_ref.shape[0]
    M1 = H1 * H1
    M2 = H2 * H2

    @pl.when(pl.program_id(0) == 0)
    def _init():
        hcp_ref[...] = jnp.zeros_like(hcp_ref)
        zqp_ref[...] = jnp.zeros_like(zqp_ref)
        gp_ref[...] = jnp.zeros_like(gp_ref)

    xs = xs_ref[0]
    for r in (0, 1):
        acc1 = jnp.zeros((H2 * H1, HID), jnp.float32) + b1_ref[...]
        for u in (0, 1):
            for v in (0, 1):
                sl = xs[u + H2 * r:u + H2 * r + H2, v:v + H1, :]
                acc1 = acc1 + _dot(sl.reshape(H2 * H1, 12), w1_ref[2 * u + v])
        h4 = jnp.maximum(acc1, 0.0).reshape(H2, H2, 2, HID)
        hcp_ref[1, 1 + H2 * r:57 + H2 * r, 0:56, :] = h4[:, :, 0, :]
        hcp_ref[0, 1 + H2 * r:57 + H2 * r, 1:57, :] = h4[:, :, 1, :]

    acc2 = jnp.zeros((M2, D), jnp.float32) + b2_ref[...]
    for ti in range(4):
        for tj in range(4):
            v = hcp_ref[tj % 2, ti:ti + 112, tj // 2:tj // 2 + H2, :]
            sl = v.reshape(H2, 2, H2, HID)[:, 0].reshape(M2, HID)
            acc2 = acc2 + _dot(sl, w2_ref[4 * ti + tj])
    z = acc2

    zz = jnp.sum(z * z, axis=1, keepdims=True)
    KC = K // 2
    mins, args = [], []
    for c in (0, 1):
        s = _dot(z, ct_ref[:, c * KC:(c + 1) * KC])
        d = (zz - 2.0 * s) + c2_ref[:, c * KC:(c + 1) * KC]
        mins.append(jnp.min(d, axis=1))
        args.append(jnp.argmin(d, axis=1))
    codes = jnp.where(mins[1] < mins[0], args[1] + KC, args[0])
    z_q = jnp.zeros((M2, D), jnp.float32)
    for c in (0, 1):
        onehot = (jax.lax.broadcasted_iota(jnp.int32, (M2, KC), 1) + c * KC
                  == codes[:, None]).astype(jnp.float32)
        z_q = z_q + _dot(onehot, cb_ref[c * KC:(c + 1) * KC, :],
                         prec=jax.lax.Precision.HIGHEST)

    zqp_ref[1:57, 1:57, :] = z_q.reshape(H2, H2, D).astype(jnp.bfloat16)
    for a in (0, 1):
        for b in (0, 1):
            accg = jnp.zeros((M2, HID), jnp.float32) + bd1_ref[...]
            for u in (0, 1):
                for v in (0, 1):
                    sl = zqp_ref[a + u:a + u + H2, b + v:b + v + H2, :]
                    accg = accg + _dot(sl.reshape(M2, D),
                                       wd1_ref[8 * a + 4 * b + 2 * u + v])
            g_ab = jnp.maximum(accg, 0.0).astype(jnp.bfloat16)
            gp_ref[a, b, 1:57, 1:57, :] = g_ab.reshape(H2, H2, HID)

    acco = jnp.zeros((M2, 48), jnp.float32) + bd2_ref[...]
    for tr in (-1, 0, 1, 2):
        for tc in (-1, 0, 1, 2):
            r0 = tr // 2 + 1
            c0 = tc // 2 + 1
            sl = gp_ref[tr % 2, tc % 2, r0:r0 + H2, c0:c0 + H2, :]
            acco = acco + _dot(sl.reshape(M2, HID),
                               wd2_ref[4 * (tr + 1) + (tc + 1)])
    o_ref[0] = acco


def kernel(x, enc_w1, enc_b1, enc_w2, enc_b2, codebook,
           dec_w1, dec_b1, dec_w2, dec_b2):
    B, Cin, H, W = x.shape
    hidden = enc_w1.shape[0]
    D = enc_w2.shape[0]
    K = codebook.shape[0]
    H1, H2 = H // 2, H // 4

    xp = jnp.pad(jnp.transpose(x, (0, 2, 3, 1)),
                 ((0, 0), (1, 1), (1, 1), (0, 0)))
    xs = xp.reshape(B, 113, 2, 113, 2, Cin).transpose(0, 1, 3, 2, 4, 5)
    xs = xs.reshape(B, 113, 113, 4 * Cin)

    w1t = enc_w1.transpose(2, 3, 1, 0)
    w1 = w1t.reshape(2, 2, 2, 2, Cin, hidden).transpose(0, 2, 1, 3, 4, 5)
    w1 = w1.reshape(4, 4 * Cin, hidden)
    w2 = enc_w2.transpose(2, 3, 1, 0).reshape(16, hidden, D)
    ct = codebook.T
    c2 = jnp.sum(codebook * codebook, axis=1).reshape(1, K)
    wd1t = dec_w1.transpose(2, 3, 1, 0)
    wd1 = jnp.stack([wd1t[a + 2 * u, b + 2 * v]
                     for a in (0, 1) for b in (0, 1)
                     for u in (0, 1) for v in (0, 1)])
    wd1 = wd1.astype(jnp.bfloat16)
    wd2t = dec_w2.transpose(2, 3, 1, 0)
    taps = []
    zeros = jnp.zeros((hidden, Cin), dec_w2.dtype)
    for tr in (-1, 0, 1, 2):
        for tc in (-1, 0, 1, 2):
            cols = []
            for ro in range(4):
                e, a = ro // 2, ro % 2
                u = tr + 1 - e - a
                for co in range(4):
                    f, b = co // 2, co % 2
                    v = tc + 1 - f - b
                    if u in (0, 1) and v in (0, 1):
                        cols.append(wd2t[a + 2 * u, b + 2 * v])
                    else:
                        cols.append(zeros)
            taps.append(jnp.concatenate(cols, axis=1))
    wd2 = jnp.stack(taps).astype(jnp.bfloat16)
    bd2 = jnp.tile(dec_b2, 16).reshape(1, 48)

    M2 = H2 * H2
    out = pl.pallas_call(
        _body,
        grid=(B,),
        in_specs=[
            pl.BlockSpec((1, 113, 113, 4 * Cin), lambda i: (i, 0, 0, 0)),
            pl.BlockSpec((4, 4 * Cin, hidden), lambda i: (0, 0, 0)),
            pl.BlockSpec((1, hidden), lambda i: (0, 0)),
            pl.BlockSpec((16, hidden, D), lambda i: (0, 0, 0)),
            pl.BlockSpec((1, D), lambda i: (0, 0)),
            pl.BlockSpec((D, K), lambda i: (0, 0)),
            pl.BlockSpec((1, K), lambda i: (0, 0)),
            pl.BlockSpec((K, D), lambda i: (0, 0)),
            pl.BlockSpec((16, D, hidden), lambda i: (0, 0, 0)),
            pl.BlockSpec((1, hidden), lambda i: (0, 0)),
            pl.BlockSpec((16, hidden, 48), lambda i: (0, 0, 0)),
            pl.BlockSpec((1, 48), lambda i: (0, 0)),
        ],
        out_specs=pl.BlockSpec((1, M2, 48), lambda i: (i, 0, 0)),
        out_shape=jax.ShapeDtypeStruct((B, M2, 48), jnp.float32),
        scratch_shapes=[
            pltpu.VMEM((2, 115, 57, hidden), jnp.float32),
            pltpu.VMEM((58, 58, D), jnp.bfloat16),
            pltpu.VMEM((2, 2, 58, 58, hidden), jnp.bfloat16),
        ],
    )(xs, w1, enc_b1.reshape(1, hidden), w2, enc_b2.reshape(1, D),
      ct, c2, codebook, wd1, dec_b1.reshape(1, hidden), wd2, bd2)

    out = out.reshape(B, H2, H2, 4, 4, Cin).transpose(0, 5, 1, 3, 2, 4)
    return out.reshape(B, Cin, H, W)
